# baseline (speedup 1.0000x reference)
import functools

import jax
import jax.numpy as jnp
from jax import lax
from jax.experimental import pallas as pl
from jax.experimental.pallas import tpu as pltpu
from jax.experimental.pallas import tpu_sc as plsc

VOCAB = 1000000
D = 64
B = 1024
K = 20
ADAPT = 64
OUT = 4096

N_TOTAL = B * (K + 1)
NC, NS = 2, 16
NW = NC * NS
ROWS_PER_W = N_TOTAL // NW
CHUNK = 112
CHUNKS = ROWS_PER_W // CHUNK

QBLK = 512
EBLK = 1024
Q_BLOCKS = B // QBLK
E_BLOCKS = (B * K) // EBLK


GROUP = 16
NGROUPS = ROWS_PER_W // GROUP


@functools.cache
def _sc_gather_fn():
    mesh = plsc.VectorSubcoreMesh(core_axis_name="c", subcore_axis_name="s")

    @functools.partial(
        pl.kernel,
        out_type=jax.ShapeDtypeStruct((N_TOTAL, D), jnp.float32),
        mesh=mesh,
        scratch_types=[
            pltpu.VMEM((ROWS_PER_W,), jnp.int32),
            pltpu.VMEM((ROWS_PER_W, D), jnp.float32),
            pltpu.SemaphoreType.DMA,
        ],
        compiler_params=pltpu.CompilerParams(use_tc_tiling_on_sc=True),
    )
    def _sc_gather(idx_hbm, table_hbm, out_hbm, idx_v, rows_v, sem):
        wid = lax.axis_index("s") * NC + lax.axis_index("c")
        pltpu.sync_copy(idx_hbm.at[wid], idx_v)

        def issue_group(g):
            vec = idx_v[pl.ds(g * GROUP, GROUP)]
            for j in range(GROUP):
                pltpu.async_copy(
                    table_hbm.at[pl.ds(vec[j], 1)],
                    rows_v.at[pl.ds(g * GROUP + j, 1)],
                    sem,
                )

        def wait_group():
            for _ in range(GROUP):
                pltpu.make_async_copy(
                    table_hbm.at[pl.ds(0, 1)], rows_v.at[pl.ds(0, 1)], sem
                ).wait()

        issue_group(0)

        def body(g, carry):
            issue_group(g)
            wait_group()
            return carry

        lax.fori_loop(1, NGROUPS, body, 0)
        wait_group()
        pltpu.sync_copy(rows_v, out_hbm.at[pl.ds(wid * ROWS_PER_W, ROWS_PER_W)])

    return _sc_gather


def _mlp_body(x_ref, w1_ref, w2_ref, o_ref):
    x = x_ref[...]
    w1 = w1_ref[...]
    w1e = w1[:, :D] + w1[:, D:]
    h = lax.dot_general(x, w1e, (((1,), (1,)), ((), ())),
                        preferred_element_type=jnp.float32)
    h = h * jax.nn.sigmoid(h)
    o_ref[...] = lax.dot_general(h, w2_ref[...], (((1,), (1,)), ((), ())),
                                 preferred_element_type=jnp.float32)


def _mlp(x, W1, W2, blk, n_blocks, row_off):
    return pl.pallas_call(
        _mlp_body,
        grid=(n_blocks,),
        in_specs=[
            pl.BlockSpec((blk, D), lambda i, o=row_off: (i + o, 0)),
            pl.BlockSpec((ADAPT, 2 * D), lambda i: (0, 0)),
            pl.BlockSpec((OUT, ADAPT), lambda i: (0, 0)),
        ],
        out_specs=pl.BlockSpec((blk, OUT), lambda i: (i, 0)),
        out_shape=jax.ShapeDtypeStruct((n_blocks * blk, OUT), jnp.float32),
        compiler_params=pltpu.CompilerParams(
            dimension_semantics=("arbitrary",)),
    )(x, W1, W2)


def kernel(query_ids, entity_ids, subgraph, emb_table, W1, W2):
    del subgraph
    flat = jnp.concatenate(
        [query_ids.astype(jnp.int32), entity_ids.reshape(-1).astype(jnp.int32)]
    )
    idx2 = flat.reshape(NW, ROWS_PER_W)
    x = _sc_gather_fn()(idx2, emb_table)
    query_embeds = _mlp(x, W1, W2, QBLK, Q_BLOCKS, 0)
    entity_embeds = _mlp(x, W1, W2, EBLK, E_BLOCKS, B // EBLK)
    return query_embeds, entity_embeds


# speedup vs baseline: 1.0024x; 1.0024x over previous
import functools

import jax
import jax.numpy as jnp
from jax import lax
from jax.experimental import pallas as pl
from jax.experimental.pallas import tpu as pltpu
from jax.experimental.pallas import tpu_sc as plsc

VOCAB = 1000000
D = 64
B = 1024
K = 20
ADAPT = 64
OUT = 4096

N_TOTAL = B * (K + 1)
NC, NS = 2, 16
NW = NC * NS

QBLK = 512
EBLK = 1024
Q_BLOCKS = B // QBLK
E_BLOCKS = (B * K) // EBLK


GROUP = 16


@functools.cache
def _sc_gather_fn(rows_per_w):
    ngroups = rows_per_w // GROUP
    n_rows = rows_per_w * NW
    mesh = plsc.VectorSubcoreMesh(core_axis_name="c", subcore_axis_name="s")

    @functools.partial(
        pl.kernel,
        out_type=jax.ShapeDtypeStruct((n_rows, D), jnp.float32),
        mesh=mesh,
        scratch_types=[
            pltpu.VMEM((rows_per_w,), jnp.int32),
            pltpu.VMEM((rows_per_w, D), jnp.float32),
            pltpu.SemaphoreType.DMA,
        ],
        compiler_params=pltpu.CompilerParams(use_tc_tiling_on_sc=True),
    )
    def _sc_gather(idx_hbm, table_hbm, out_hbm, idx_v, rows_v, sem):
        wid = lax.axis_index("s") * NC + lax.axis_index("c")
        pltpu.sync_copy(idx_hbm.at[wid], idx_v)

        def issue_group(g):
            vec = idx_v[pl.ds(g * GROUP, GROUP)]
            for j in range(GROUP):
                pltpu.async_copy(
                    table_hbm.at[pl.ds(vec[j], 1)],
                    rows_v.at[pl.ds(g * GROUP + j, 1)],
                    sem,
                )

        def wait_group():
            for _ in range(GROUP):
                pltpu.make_async_copy(
                    table_hbm.at[pl.ds(0, 1)], rows_v.at[pl.ds(0, 1)], sem
                ).wait()

        issue_group(0)

        def body(g, carry):
            issue_group(g)
            wait_group()
            return carry

        lax.fori_loop(1, ngroups, body, 0)
        wait_group()
        pltpu.sync_copy(rows_v, out_hbm.at[pl.ds(wid * rows_per_w, rows_per_w)])

    return _sc_gather


def _mlp_body(x_ref, w1_ref, w2_ref, o_ref):
    x = x_ref[...]
    w1 = w1_ref[...]
    w1e = w1[:, :D] + w1[:, D:]
    h = lax.dot_general(x, w1e, (((1,), (1,)), ((), ())),
                        preferred_element_type=jnp.float32)
    h = h * jax.nn.sigmoid(h)
    o_ref[...] = lax.dot_general(h, w2_ref[...], (((1,), (1,)), ((), ())),
                                 preferred_element_type=jnp.float32)


def _mlp(x, W1, W2, blk, n_blocks, row_off):
    return pl.pallas_call(
        _mlp_body,
        grid=(n_blocks,),
        in_specs=[
            pl.BlockSpec((blk, D), lambda i, o=row_off: (i + o, 0)),
            pl.BlockSpec((ADAPT, 2 * D), lambda i: (0, 0)),
            pl.BlockSpec((OUT, ADAPT), lambda i: (0, 0)),
        ],
        out_specs=pl.BlockSpec((blk, OUT), lambda i: (i, 0)),
        out_shape=jax.ShapeDtypeStruct((n_blocks * blk, OUT), jnp.float32),
        compiler_params=pltpu.CompilerParams(
            dimension_semantics=("arbitrary",)),
    )(x, W1, W2)


def kernel(query_ids, entity_ids, subgraph, emb_table, W1, W2):
    del subgraph
    idx_q = query_ids.astype(jnp.int32).reshape(NW, B // NW)
    idx_e = entity_ids.astype(jnp.int32).reshape(NW, (B * K) // NW)
    xq = _sc_gather_fn(B // NW)(idx_q, emb_table)
    xe = _sc_gather_fn((B * K) // NW)(idx_e, emb_table)
    query_embeds = _mlp(xq, W1, W2, QBLK, Q_BLOCKS, 0)
    entity_embeds = _mlp(xe, W1, W2, EBLK, E_BLOCKS, 0)
    return query_embeds, entity_embeds


# speedup vs baseline: 1.0042x; 1.0018x over previous
import functools

import jax
import jax.numpy as jnp
from jax import lax
from jax.experimental import pallas as pl
from jax.experimental.pallas import tpu as pltpu
from jax.experimental.pallas import tpu_sc as plsc

VOCAB = 1000000
D = 64
B = 1024
K = 20
ADAPT = 64
OUT = 4096

N_TOTAL = B * (K + 1)
NC, NS = 2, 16
NW = NC * NS

QBLK = 512
EBLK = 1024
Q_BLOCKS = B // QBLK
E_BLOCKS = (B * K) // EBLK


GROUP = 16


@functools.cache
def _sc_gather_fn(rows_per_w):
    ngroups = rows_per_w // GROUP
    n_rows = rows_per_w * NW
    mesh = plsc.VectorSubcoreMesh(core_axis_name="c", subcore_axis_name="s")

    @functools.partial(
        pl.kernel,
        out_type=jax.ShapeDtypeStruct((n_rows, D), jnp.float32),
        mesh=mesh,
        scratch_types=[
            pltpu.VMEM((rows_per_w,), jnp.int32),
            pltpu.VMEM((rows_per_w, D), jnp.float32),
            pltpu.SemaphoreType.DMA,
        ],
    )
    def _sc_gather(idx_hbm, table_hbm, out_hbm, idx_v, rows_v, sem):
        wid = lax.axis_index("s") * NC + lax.axis_index("c")
        pltpu.sync_copy(idx_hbm.at[wid], idx_v)

        def issue_group(g):
            vec = idx_v[pl.ds(g * GROUP, GROUP)]
            for j in range(GROUP):
                pltpu.async_copy(
                    table_hbm.at[pl.ds(vec[j], 1)],
                    rows_v.at[pl.ds(g * GROUP + j, 1)],
                    sem,
                )

        def wait_group():
            for _ in range(GROUP):
                pltpu.make_async_copy(
                    table_hbm.at[pl.ds(0, 1)], rows_v.at[pl.ds(0, 1)], sem
                ).wait()

        issue_group(0)

        def body(g, carry):
            issue_group(g)
            wait_group()
            return carry

        lax.fori_loop(1, ngroups, body, 0)
        wait_group()
        pltpu.sync_copy(rows_v, out_hbm.at[pl.ds(wid * rows_per_w, rows_per_w)])

    return _sc_gather


def _mlp_body(x_ref, w1_ref, w2_ref, o_ref):
    x = x_ref[...]
    w1 = w1_ref[...]
    w1e = w1[:, :D] + w1[:, D:]
    h = lax.dot_general(x, w1e, (((1,), (1,)), ((), ())),
                        preferred_element_type=jnp.float32)
    h = h * jax.nn.sigmoid(h)
    o_ref[...] = lax.dot_general(h, w2_ref[...], (((1,), (1,)), ((), ())),
                                 preferred_element_type=jnp.float32)


def _mlp(x, W1, W2, blk, n_blocks, row_off):
    return pl.pallas_call(
        _mlp_body,
        grid=(n_blocks,),
        in_specs=[
            pl.BlockSpec((blk, D), lambda i, o=row_off: (i + o, 0)),
            pl.BlockSpec((ADAPT, 2 * D), lambda i: (0, 0)),
            pl.BlockSpec((OUT, ADAPT), lambda i: (0, 0)),
        ],
        out_specs=pl.BlockSpec((blk, OUT), lambda i: (i, 0)),
        out_shape=jax.ShapeDtypeStruct((n_blocks * blk, OUT), jnp.float32),
        compiler_params=pltpu.CompilerParams(
            dimension_semantics=("arbitrary",)),
    )(x, W1, W2)


def kernel(query_ids, entity_ids, subgraph, emb_table, W1, W2):
    del subgraph
    idx_q = query_ids.astype(jnp.int32).reshape(NW, B // NW)
    idx_e = entity_ids.astype(jnp.int32).reshape(NW, (B * K) // NW)
    xq = _sc_gather_fn(B // NW)(idx_q, emb_table)
    xe = _sc_gather_fn((B * K) // NW)(idx_e, emb_table)
    query_embeds = _mlp(xq, W1, W2, QBLK, Q_BLOCKS, 0)
    entity_embeds = _mlp(xe, W1, W2, EBLK, E_BLOCKS, 0)
    return query_embeds, entity_embeds


# speedup vs baseline: 1.1739x; 1.1690x over previous
import functools

import jax
import jax.numpy as jnp
from jax import lax
from jax.experimental import pallas as pl
from jax.experimental.pallas import tpu as pltpu
from jax.experimental.pallas import tpu_sc as plsc

VOCAB = 1000000
D = 64
B = 1024
K = 20
ADAPT = 64
OUT = 4096

N_TOTAL = B * (K + 1)
NC, NS = 2, 16
NW = NC * NS

QBLK = 512
EBLK = 1024
Q_BLOCKS = B // QBLK
E_BLOCKS = (B * K) // EBLK

VB = 8192

GROUP = 16


def _relayout_body(xt_ref, o_ref):
    o_ref[...] = xt_ref[...].T


def _relayout(tT):
    nb = pl.cdiv(VOCAB, VB)
    return pl.pallas_call(
        _relayout_body,
        grid=(nb,),
        in_specs=[pl.BlockSpec((D, VB), lambda i: (0, i))],
        out_specs=pl.BlockSpec((VB, D), lambda i: (i, 0)),
        out_shape=jax.ShapeDtypeStruct((VOCAB, D), jnp.float32),
        compiler_params=pltpu.CompilerParams(
            dimension_semantics=("arbitrary",)),
    )(tT)


@functools.cache
def _sc_gather_fn(rows_per_w):
    ngroups = rows_per_w // GROUP
    n_rows = rows_per_w * NW
    mesh = plsc.VectorSubcoreMesh(core_axis_name="c", subcore_axis_name="s")

    @functools.partial(
        pl.kernel,
        out_type=jax.ShapeDtypeStruct((n_rows, D), jnp.float32),
        mesh=mesh,
        scratch_types=[
            pltpu.VMEM((rows_per_w,), jnp.int32),
            pltpu.VMEM((rows_per_w, D), jnp.float32),
            pltpu.SemaphoreType.DMA,
        ],
        compiler_params=pltpu.CompilerParams(use_tc_tiling_on_sc=True),
    )
    def _sc_gather(idx_hbm, table_hbm, out_hbm, idx_v, rows_v, sem):
        wid = lax.axis_index("s") * NC + lax.axis_index("c")
        pltpu.sync_copy(idx_hbm.at[wid], idx_v)

        def issue_group(g):
            vec = idx_v[pl.ds(g * GROUP, GROUP)]
            for j in range(GROUP):
                pltpu.async_copy(
                    table_hbm.at[pl.ds(vec[j], 1)],
                    rows_v.at[pl.ds(g * GROUP + j, 1)],
                    sem,
                )

        def wait_group():
            for _ in range(GROUP):
                pltpu.make_async_copy(
                    table_hbm.at[pl.ds(0, 1)], rows_v.at[pl.ds(0, 1)], sem
                ).wait()

        issue_group(0)

        def body(g, carry):
            issue_group(g)
            wait_group()
            return carry

        lax.fori_loop(1, ngroups, body, 0)
        wait_group()
        pltpu.sync_copy(rows_v, out_hbm.at[pl.ds(wid * rows_per_w, rows_per_w)])

    return _sc_gather


def _mlp_body(x_ref, w1_ref, w2_ref, o_ref):
    x = x_ref[...]
    w1 = w1_ref[...]
    w1e = w1[:, :D] + w1[:, D:]
    h = lax.dot_general(x, w1e, (((1,), (1,)), ((), ())),
                        preferred_element_type=jnp.float32)
    h = h * jax.nn.sigmoid(h)
    o_ref[...] = lax.dot_general(h, w2_ref[...], (((1,), (1,)), ((), ())),
                                 preferred_element_type=jnp.float32)


def _mlp(x, W1, W2, blk, n_blocks):
    return pl.pallas_call(
        _mlp_body,
        grid=(n_blocks,),
        in_specs=[
            pl.BlockSpec((blk, D), lambda i: (i, 0)),
            pl.BlockSpec((ADAPT, 2 * D), lambda i: (0, 0)),
            pl.BlockSpec((OUT, ADAPT), lambda i: (0, 0)),
        ],
        out_specs=pl.BlockSpec((blk, OUT), lambda i: (i, 0)),
        out_shape=jax.ShapeDtypeStruct((n_blocks * blk, OUT), jnp.float32),
        compiler_params=pltpu.CompilerParams(
            dimension_semantics=("arbitrary",)),
    )(x, W1, W2)


def kernel(query_ids, entity_ids, subgraph, emb_table, W1, W2):
    del subgraph
    idx_q = query_ids.astype(jnp.int32).reshape(NW, B // NW)
    idx_e = entity_ids.astype(jnp.int32).reshape(NW, (B * K) // NW)
    tbl = _relayout(emb_table.T)
    xq = _sc_gather_fn(B // NW)(idx_q, tbl)
    xe = _sc_gather_fn((B * K) // NW)(idx_e, tbl)
    query_embeds = _mlp(xq, W1, W2, QBLK, Q_BLOCKS)
    entity_embeds = _mlp(xe, W1, W2, EBLK, E_BLOCKS)
    return query_embeds, entity_embeds
